# baseline (speedup 1.0000x reference)
import functools

import jax
import jax.numpy as jnp
from jax import lax
from jax.experimental import pallas as pl
from jax.experimental.pallas import tpu as pltpu
from jax.experimental.pallas import tpu_sc as plsc

VOCAB = 1000000
DIM = 128
RANK = 32
N_TOK = 4096 * 200
N4 = N_TOK // 4

_info = plsc.get_sparse_core_info()
NC, NS = _info.num_cores, _info.num_subcores
NW = NC * NS
N_PER_W = N4 // NW
CHUNK4 = 400
N_CHUNKS = N_PER_W // CHUNK4

_sc_mesh = plsc.VectorSubcoreMesh(core_axis_name="c", subcore_axis_name="s")


@functools.partial(
    pl.kernel,
    mesh=_sc_mesh,
    out_type=jax.ShapeDtypeStruct((N4, 128), jnp.float32),
    scratch_types=[
        pltpu.VMEM((4, CHUNK4), jnp.int32),
        pltpu.VMEM((4 * CHUNK4, RANK), jnp.float32),
        pltpu.SemaphoreType.DMA,
    ],
    compiler_params=pltpu.CompilerParams(use_tc_tiling_on_sc=False),
)
def _sc_gather(table_hbm, idx_hbm, out_hbm, idx_v, rows_v, sem):
    wid = lax.axis_index("s") * NC + lax.axis_index("c")
    base = wid * N_PER_W

    def chunk_body(c, carry):
        off = base + c * CHUNK4
        pltpu.sync_copy(idx_hbm.at[:, pl.ds(off, CHUNK4)], idx_v)
        copies = [
            pltpu.async_copy(
                table_hbm.at[idx_v.at[a]],
                rows_v.at[pl.ds(CHUNK4 * a, CHUNK4)],
                sem,
            )
            for a in range(4)
        ]
        for cp in copies:
            cp.wait()
        for a in range(4):
            pltpu.sync_copy(
                rows_v.at[pl.ds(CHUNK4 * a, CHUNK4)],
                out_hbm.at[pl.ds(off, CHUNK4), pl.ds(RANK * a, RANK)],
            )
        return carry

    lax.fori_loop(0, N_CHUNKS, chunk_body, 0)


BLK = 8192
BLK4 = BLK // 4


def _mm_body(u4_ref, v_ref, o_ref):
    u4 = u4_ref[...]
    v = v_ref[...]
    ps = [
        jnp.dot(u4[:, RANK * a:RANK * (a + 1)], v,
                preferred_element_type=jnp.float32)
        for a in range(4)
    ]
    o_ref[...] = jnp.stack(ps, axis=1).reshape(BLK, DIM)


def kernel(token_ids, U, V):
    B, S = token_ids.shape
    ids4 = jnp.swapaxes(
        token_ids.reshape(-1).astype(jnp.int32).reshape(N4, 4), 0, 1)
    u4 = _sc_gather(U, ids4)

    out = pl.pallas_call(
        _mm_body,
        grid=(N_TOK // BLK,),
        in_specs=[
            pl.BlockSpec((BLK4, 128), lambda i: (i, 0)),
            pl.BlockSpec((RANK, DIM), lambda i: (0, 0)),
        ],
        out_specs=pl.BlockSpec((BLK, DIM), lambda i: (i, 0)),
        out_shape=jax.ShapeDtypeStruct((N_TOK, DIM), jnp.float32),
    )(u4, V)
    return out.reshape(B, S, DIM)


# speedup vs baseline: 1.2933x; 1.2933x over previous
import functools

import jax
import jax.numpy as jnp
from jax import lax
from jax.experimental import pallas as pl
from jax.experimental.pallas import tpu as pltpu
from jax.experimental.pallas import tpu_sc as plsc

VOCAB = 1000000
DIM = 128
RANK = 32
BATCH = 4096
SEQ = 200
N_TOK = BATCH * SEQ
N4 = N_TOK // 4

_info = plsc.get_sparse_core_info()
NC, NS = _info.num_cores, _info.num_subcores
NW = NC * NS
TOK_PER_W = N_TOK // NW
CHUNK_T = 1600
CHUNK_L = CHUNK_T // 4
N_CHUNKS = TOK_PER_W // CHUNK_T

_sc_mesh = plsc.VectorSubcoreMesh(core_axis_name="c", subcore_axis_name="s")


@functools.partial(
    pl.kernel,
    mesh=_sc_mesh,
    out_type=jax.ShapeDtypeStruct((N4, 128), jnp.float32),
    scratch_types=[
        pltpu.VMEM((CHUNK_T,), jnp.int32),
        pltpu.VMEM((CHUNK_T, RANK), jnp.float32),
        pltpu.SemaphoreType.DMA,
    ],
    compiler_params=pltpu.CompilerParams(use_tc_tiling_on_sc=False),
)
def _sc_gather(table_hbm, idx_hbm, out_hbm, idx_v, rows_v, sem):
    wid = lax.axis_index("s") * NC + lax.axis_index("c")
    tbase = wid * TOK_PER_W
    lbase = wid * (TOK_PER_W // 4)

    def chunk_body(c, carry):
        toff = tbase + c * CHUNK_T
        loff = lbase + c * CHUNK_L
        pltpu.sync_copy(idx_hbm.at[pl.ds(toff, CHUNK_T)], idx_v)
        copies = [
            pltpu.async_copy(
                table_hbm.at[idx_v.at[pl.ds(CHUNK_L * a, CHUNK_L)]],
                rows_v.at[pl.ds(CHUNK_L * a, CHUNK_L)],
                sem,
            )
            for a in range(4)
        ]
        for cp in copies:
            cp.wait()
        for a in range(4):
            pltpu.sync_copy(
                rows_v.at[pl.ds(CHUNK_L * a, CHUNK_L)],
                out_hbm.at[pl.ds(loff, CHUNK_L), pl.ds(RANK * a, RANK)],
            )
        return carry

    lax.fori_loop(0, N_CHUNKS, chunk_body, 0)


CH_PER_STEP = 4
STEP_L = CH_PER_STEP * CHUNK_L
STEP_T = CH_PER_STEP * CHUNK_T
STEP_B = STEP_T // SEQ


def _mm_body(u4_ref, w_ref, o_ref):
    w = w_ref[...]
    outs = []
    for g in range(CH_PER_STEP):
        p = jnp.dot(u4_ref[pl.ds(CHUNK_L * g, CHUNK_L), :], w,
                    preferred_element_type=jnp.float32)
        outs.extend(p[:, DIM * a:DIM * (a + 1)] for a in range(4))
    o_ref[...] = jnp.concatenate(outs, axis=0).reshape(STEP_B, SEQ, DIM)


def kernel(token_ids, U, V):
    ids = token_ids.reshape(-1).astype(jnp.int32)
    u4 = _sc_gather(U, ids)
    w = jnp.kron(jnp.eye(4, dtype=jnp.float32), V)

    out = pl.pallas_call(
        _mm_body,
        grid=(N_TOK // STEP_T,),
        in_specs=[
            pl.BlockSpec((STEP_L, 128), lambda i: (i, 0)),
            pl.BlockSpec((128, 4 * DIM), lambda i: (0, 0)),
        ],
        out_specs=pl.BlockSpec((STEP_B, SEQ, DIM), lambda i: (i, 0, 0)),
        out_shape=jax.ShapeDtypeStruct((BATCH, SEQ, DIM), jnp.float32),
    )(u4, w)
    return out


# speedup vs baseline: 1.3402x; 1.0362x over previous
import functools

import jax
import jax.numpy as jnp
from jax import lax
from jax.experimental import pallas as pl
from jax.experimental.pallas import tpu as pltpu
from jax.experimental.pallas import tpu_sc as plsc

VOCAB = 1000000
DIM = 128
RANK = 32
BATCH = 4096
SEQ = 200
N_TOK = BATCH * SEQ
N4 = N_TOK // 4

_info = plsc.get_sparse_core_info()
NC, NS = _info.num_cores, _info.num_subcores
NW = NC * NS
TOK_PER_W = N_TOK // NW
CHUNK_T = 1600
CHUNK_L = CHUNK_T // 4
N_CHUNKS = TOK_PER_W // CHUNK_T

_sc_mesh = plsc.VectorSubcoreMesh(core_axis_name="c", subcore_axis_name="s")


@functools.partial(
    pl.kernel,
    mesh=_sc_mesh,
    out_type=jax.ShapeDtypeStruct((N4, 128), jnp.float32),
    scratch_types=[
        pltpu.VMEM((CHUNK_T,), jnp.int32),
        pltpu.VMEM((CHUNK_T, RANK), jnp.float32),
        pltpu.SemaphoreType.DMA,
    ],
    compiler_params=pltpu.CompilerParams(use_tc_tiling_on_sc=False),
)
def _sc_gather(table_hbm, idx_hbm, out_hbm, idx_v, rows_v, sem):
    wid = lax.axis_index("s") * NC + lax.axis_index("c")
    tbase = wid * TOK_PER_W
    lbase = wid * (TOK_PER_W // 4)

    def chunk_body(c, carry):
        toff = tbase + c * CHUNK_T
        loff = lbase + c * CHUNK_L
        pltpu.sync_copy(idx_hbm.at[pl.ds(toff, CHUNK_T)], idx_v)
        copies = [
            pltpu.async_copy(
                table_hbm.at[idx_v.at[pl.ds(CHUNK_L * a, CHUNK_L)]],
                rows_v.at[pl.ds(CHUNK_L * a, CHUNK_L)],
                sem,
            )
            for a in range(4)
        ]
        for cp in copies:
            cp.wait()
        for a in range(4):
            pltpu.sync_copy(
                rows_v.at[pl.ds(CHUNK_L * a, CHUNK_L)],
                out_hbm.at[pl.ds(loff, CHUNK_L), pl.ds(RANK * a, RANK)],
            )
        return carry

    lax.fori_loop(0, N_CHUNKS, chunk_body, 0)


CH_PER_STEP = 8
STEP_L = CH_PER_STEP * CHUNK_L
STEP_T = CH_PER_STEP * CHUNK_T
STEP_B = STEP_T // SEQ


def _mm_body(u4_ref, w_ref, o_ref):
    w = w_ref[...]
    outs = []
    for g in range(CH_PER_STEP):
        p = jnp.dot(u4_ref[pl.ds(CHUNK_L * g, CHUNK_L), :], w,
                    preferred_element_type=jnp.float32)
        outs.extend(p[:, DIM * a:DIM * (a + 1)] for a in range(4))
    o_ref[...] = jnp.concatenate(outs, axis=0).reshape(STEP_B, SEQ, DIM)


def kernel(token_ids, U, V):
    ids = token_ids.reshape(-1).astype(jnp.int32)
    u4 = _sc_gather(U, ids)
    w = jnp.kron(jnp.eye(4, dtype=jnp.float32), V)

    out = pl.pallas_call(
        _mm_body,
        grid=(N_TOK // STEP_T,),
        in_specs=[
            pl.BlockSpec((STEP_L, 128), lambda i: (i, 0)),
            pl.BlockSpec((128, 4 * DIM), lambda i: (0, 0)),
        ],
        out_specs=pl.BlockSpec((STEP_B, SEQ, DIM), lambda i: (i, 0, 0)),
        out_shape=jax.ShapeDtypeStruct((BATCH, SEQ, DIM), jnp.float32),
    )(u4, w)
    return out


# speedup vs baseline: 1.3520x; 1.0088x over previous
import functools

import jax
import jax.numpy as jnp
from jax import lax
from jax.experimental import pallas as pl
from jax.experimental.pallas import tpu as pltpu
from jax.experimental.pallas import tpu_sc as plsc

VOCAB = 1000000
DIM = 128
RANK = 32
BATCH = 4096
SEQ = 200
N_TOK = BATCH * SEQ
N4 = N_TOK // 4

_info = plsc.get_sparse_core_info()
NC, NS = _info.num_cores, _info.num_subcores
NW = NC * NS
TOK_PER_W = N_TOK // NW
CHUNK_T = 1600
CHUNK_L = CHUNK_T // 4
N_CHUNKS = TOK_PER_W // CHUNK_T

_sc_mesh = plsc.VectorSubcoreMesh(core_axis_name="c", subcore_axis_name="s")


@functools.partial(
    pl.kernel,
    mesh=_sc_mesh,
    out_type=jax.ShapeDtypeStruct((N4, 128), jnp.float32),
    scratch_types=[
        pltpu.VMEM((CHUNK_T,), jnp.int32),
        pltpu.VMEM((CHUNK_T, RANK), jnp.float32),
        pltpu.SemaphoreType.DMA,
    ],
    compiler_params=pltpu.CompilerParams(use_tc_tiling_on_sc=False),
)
def _sc_gather(table_hbm, idx_hbm, out_hbm, idx_v, rows_v, sem):
    wid = lax.axis_index("s") * NC + lax.axis_index("c")
    tbase = wid * TOK_PER_W
    lbase = wid * (TOK_PER_W // 4)

    def chunk_body(c, carry):
        toff = tbase + c * CHUNK_T
        loff = lbase + c * CHUNK_L
        pltpu.sync_copy(idx_hbm.at[pl.ds(toff, CHUNK_T)], idx_v)
        copies = [
            pltpu.async_copy(
                table_hbm.at[idx_v.at[pl.ds(CHUNK_L * a, CHUNK_L)]],
                rows_v.at[pl.ds(CHUNK_L * a, CHUNK_L)],
                sem,
            )
            for a in range(4)
        ]
        for cp in copies:
            cp.wait()
        for a in range(4):
            pltpu.sync_copy(
                rows_v.at[pl.ds(CHUNK_L * a, CHUNK_L)],
                out_hbm.at[pl.ds(loff, CHUNK_L), pl.ds(RANK * a, RANK)],
            )
        return carry

    lax.fori_loop(0, N_CHUNKS, chunk_body, 0)


CH_PER_STEP = 16
STEP_L = CH_PER_STEP * CHUNK_L
STEP_T = CH_PER_STEP * CHUNK_T
STEP_B = STEP_T // SEQ


def _mm_body(u4_ref, w_ref, o_ref):
    w = w_ref[...]
    outs = []
    for g in range(CH_PER_STEP):
        p = jnp.dot(u4_ref[pl.ds(CHUNK_L * g, CHUNK_L), :], w,
                    preferred_element_type=jnp.float32)
        outs.extend(p[:, DIM * a:DIM * (a + 1)] for a in range(4))
    o_ref[...] = jnp.concatenate(outs, axis=0).reshape(STEP_B, SEQ, DIM)


def kernel(token_ids, U, V):
    ids = token_ids.reshape(-1).astype(jnp.int32)
    u4 = _sc_gather(U, ids)
    w = jnp.kron(jnp.eye(4, dtype=jnp.float32), V)

    out = pl.pallas_call(
        _mm_body,
        grid=(N_TOK // STEP_T,),
        in_specs=[
            pl.BlockSpec((STEP_L, 128), lambda i: (i, 0)),
            pl.BlockSpec((128, 4 * DIM), lambda i: (0, 0)),
        ],
        out_specs=pl.BlockSpec((STEP_B, SEQ, DIM), lambda i: (i, 0, 0)),
        out_shape=jax.ShapeDtypeStruct((BATCH, SEQ, DIM), jnp.float32),
    )(u4, w)
    return out
